# baseline (speedup 1.0000x reference)
import jax
import jax.numpy as jnp
from jax import lax
from jax.experimental import pallas as pl
from jax.experimental.pallas import tpu as pltpu


def _l2n(x):
    eps = jnp.float32(1e-12)
    denom = jnp.maximum(jnp.sqrt(jnp.sum(x * x, axis=-1, keepdims=True)), eps)
    return x * (jnp.float32(1.0) / denom)


def _align_kernel(img_ref, txt_ref, out_ref):
    bs = img_ref.shape[0]
    for s in range(bs):
        img = img_ref[s]
        txt = txt_ref[s]

        img_n = _l2n(img)
        txt_n = _l2n(txt)

        sim = lax.dot_general(
            img_n, txt_n, (((1,), (1,)), ((), ())),
            preferred_element_type=jnp.float32)

        n_img, n_txt = sim.shape
        row_max = jnp.max(sim, axis=1, keepdims=True)
        colf = lax.broadcasted_iota(
            jnp.int32, (n_img, n_txt), 1).astype(jnp.float32)
        idxf = jnp.min(jnp.where(sim == row_max, colf, jnp.float32(n_txt)),
                       axis=1, keepdims=True)

        one_hot = (colf == idxf).astype(jnp.float32)
        out_ref[s] = lax.dot_general(
            one_hot, txt, (((1,), (0,)), ((), ())),
            preferred_element_type=jnp.float32)


def kernel(img_feat, text_feat):
    B, N_img, C = img_feat.shape
    _, N_txt, _ = text_feat.shape
    BS = 4
    return pl.pallas_call(
        _align_kernel,
        grid=(B // BS,),
        in_specs=[
            pl.BlockSpec((BS, N_img, C), lambda b: (b, 0, 0)),
            pl.BlockSpec((BS, N_txt, C), lambda b: (b, 0, 0)),
        ],
        out_specs=pl.BlockSpec((BS, N_img, C), lambda b: (b, 0, 0)),
        out_shape=jax.ShapeDtypeStruct((B, N_img, C), jnp.float32),
    )(img_feat, text_feat)
